# baseline (speedup 1.0000x reference)
import functools

import jax
import jax.numpy as jnp
from jax import lax
from jax.experimental import pallas as pl
from jax.experimental.pallas import tpu as pltpu
from jax.experimental.pallas import tpu_sc as plsc

D = 128
LANES = 16
NTILES = 16
CHUNK = 128
GCHUNKS = 8
GEDGES = GCHUNKS * CHUNK


def _round_up(v, m):
    return (v + m - 1) // m * m


@functools.lru_cache(maxsize=None)
def _make_sc_segsum(n_pad, e_per_worker, full_range):
    nh = n_pad if full_range else n_pad // 2
    acc_rows = nh + 8
    nzc = nh // CHUNK
    rpt = nh // NTILES
    nchunks = e_per_worker // CHUNK
    out_shape = ((2, n_pad, D) if full_range else (n_pad, D))
    mesh = plsc.VectorSubcoreMesh(core_axis_name="c", subcore_axis_name="s")

    @functools.partial(
        pl.kernel,
        mesh=mesh,
        out_type=jax.ShapeDtypeStruct(out_shape, jnp.float32),
        scratch_types=[
            pltpu.VMEM((e_per_worker,), jnp.int32),
            pltpu.VMEM((e_per_worker,), jnp.int32),
            pltpu.VMEM((CHUNK,), jnp.int32),
            pltpu.VMEM((CHUNK, D), jnp.float32),
            pltpu.VMEM_SHARED((acc_rows, D), jnp.float32),
            pltpu.SemaphoreType.DMA,
        ],
    )
    def segsum(h_hbm, src_hbm, dst_hbm, out_hbm, src_v, dst_v, didx, rows_v,
               acc, sem):
        c = lax.axis_index("c")
        s = lax.axis_index("s")
        base = c * nh

        def zero_row(r, carry):
            for i in range(D // LANES):
                rows_v[r, pl.ds(i * LANES, LANES)] = jnp.zeros(
                    (LANES,), jnp.float32)
            return carry

        lax.fori_loop(0, CHUNK, zero_row, 0)

        def zero_acc(jj, carry):
            j = s + jj * NTILES

            @pl.when(j < nzc)
            def _():
                pltpu.sync_copy(rows_v, acc.at[pl.ds(j * CHUNK, CHUNK)])

            return carry

        lax.fori_loop(0, (nzc + NTILES - 1) // NTILES, zero_acc, 0)

        w = (c * NTILES + s) if full_range else s
        ebase = w * e_per_worker
        pltpu.sync_copy(src_hbm.at[pl.ds(ebase, e_per_worker)], src_v)
        pltpu.sync_copy(dst_hbm.at[pl.ds(ebase, e_per_worker)], dst_v)
        plsc.subcore_barrier()

        def chunk_body(ch, carry):
            eoff = ch * CHUNK
            pltpu.async_copy(
                h_hbm.at[src_v.at[pl.ds(eoff, CHUNK)]], rows_v, sem).wait()
            for i in range(CHUNK // LANES):
                d_vec = dst_v[pl.ds(eoff + i * LANES, LANES)]
                if full_range:
                    didx[pl.ds(i * LANES, LANES)] = d_vec
                else:
                    dl = d_vec - base
                    ok = (dl >= 0) & (dl < nh)
                    didx[pl.ds(i * LANES, LANES)] = jnp.where(ok, dl, nh)
            pltpu.sync_copy(rows_v, acc.at[didx], add=True)
            return carry

        lax.fori_loop(0, nchunks, chunk_body, 0)
        plsc.subcore_barrier()

        if full_range:
            pltpu.sync_copy(acc.at[pl.ds(s * rpt, rpt)],
                            out_hbm.at[c, pl.ds(s * rpt, rpt)])
        else:
            pltpu.sync_copy(acc.at[pl.ds(s * rpt, rpt)],
                            out_hbm.at[pl.ds(c * nh + s * rpt, rpt)])

    return segsum


def _gin_mlp(h, aggs, w1, b1, w2, b2, eps, relu_out, block_m):
    n_pad = h.shape[0]
    d_h = w1.shape[1]
    naggs = len(aggs)

    def body(h_ref, *refs):
        a_refs = refs[:naggs]
        w1_ref, b1_ref, w2_ref, b2_ref, e_ref, o_ref = refs[naggs:]
        z = (1.0 + e_ref[0, 0]) * h_ref[...]
        for a_ref in a_refs:
            z = z + a_ref[...]
        z1 = jnp.dot(z, w1_ref[...], preferred_element_type=jnp.float32)
        z1 = jnp.maximum(z1 + b1_ref[...], 0.0)
        z2 = jnp.dot(z1, w2_ref[...], preferred_element_type=jnp.float32)
        z2 = z2 + b2_ref[...]
        if relu_out:
            z2 = jnp.maximum(z2, 0.0)
        o_ref[...] = z2

    return pl.pallas_call(
        body,
        grid=(n_pad // block_m,),
        in_specs=[pl.BlockSpec((block_m, D), lambda i: (i, 0))] * (1 + naggs)
        + [
            pl.BlockSpec((D, d_h), lambda i: (0, 0)),
            pl.BlockSpec((1, d_h), lambda i: (0, 0)),
            pl.BlockSpec((d_h, D), lambda i: (0, 0)),
            pl.BlockSpec((1, D), lambda i: (0, 0)),
            pl.BlockSpec((1, 1), lambda i: (0, 0)),
        ],
        out_specs=pl.BlockSpec((block_m, D), lambda i: (i, 0)),
        out_shape=jax.ShapeDtypeStruct((n_pad, D), jnp.float32),
    )(h, *aggs, w1, b1, w2, b2, eps)


def _mean_pool(h, seg2d, nseg, block_m):
    n_pad = h.shape[0]
    nsteps = n_pad // block_m

    def body(h_ref, s_ref, sum_ref, cnt_ref):
        i = pl.program_id(0)

        @pl.when(i == 0)
        def _():
            sum_ref[...] = jnp.zeros_like(sum_ref)
            cnt_ref[...] = jnp.zeros_like(cnt_ref)

        onehot = (s_ref[...] == lax.broadcasted_iota(
            jnp.int32, (block_m, nseg), 1)).astype(jnp.float32)
        sum_ref[...] += lax.dot_general(
            onehot, h_ref[...], (((0,), (0,)), ((), ())),
            preferred_element_type=jnp.float32)
        cnt_ref[...] += lax.dot_general(
            onehot, jnp.ones((block_m, 1), jnp.float32),
            (((0,), (0,)), ((), ())), preferred_element_type=jnp.float32)

        @pl.when(i == nsteps - 1)
        def _():
            sum_ref[...] = sum_ref[...] / jnp.maximum(cnt_ref[...], 1.0)

    mean, _ = pl.pallas_call(
        body,
        grid=(nsteps,),
        in_specs=[
            pl.BlockSpec((block_m, D), lambda i: (i, 0)),
            pl.BlockSpec((block_m, 1), lambda i: (i, 0)),
        ],
        out_specs=[
            pl.BlockSpec((nseg, D), lambda i: (0, 0)),
            pl.BlockSpec((nseg, 1), lambda i: (0, 0)),
        ],
        out_shape=[
            jax.ShapeDtypeStruct((nseg, D), jnp.float32),
            jax.ShapeDtypeStruct((nseg, 1), jnp.float32),
        ],
    )(h, seg2d)
    return mean


def _tail(hg, hs, smf, mw1, mb1, mw2, mb2, cw, cb, threshold, margin):
    b, s = smf.shape

    def body(hg_ref, hs_ref, smf_ref, mw1_ref, mb1_ref, mw2_ref, mb2_ref,
             cw_ref, cb_ref, logits_ref, closs_ref, sg_ref):
        hs_v = hs_ref[...]
        m1 = jnp.dot(hs_v, mw1_ref[...], preferred_element_type=jnp.float32)
        m1 = jnp.maximum(m1 + mb1_ref[...], 0.0)
        m = jnp.dot(m1, mw2_ref[...], preferred_element_type=jnp.float32)
        m = m + mb2_ref[...]
        sg = 1.0 / (1.0 + jnp.exp(-m))
        sg_ref[...] = sg

        smf_v = smf_ref[...]
        vmask = (sg > threshold).astype(jnp.float32)
        emask = (sg <= threshold - 0.1).astype(jnp.float32)
        ha_num = jnp.dot(smf_v, vmask * hs_v,
                         preferred_element_type=jnp.float32)
        da = jnp.dot(smf_v, vmask, preferred_element_type=jnp.float32)
        ha = ha_num / jnp.maximum(da, 1.0)
        he_num = jnp.dot(smf_v, emask * hs_v,
                         preferred_element_type=jnp.float32)
        de = jnp.dot(smf_v, emask, preferred_element_type=jnp.float32)
        he = he_num / jnp.maximum(de, 1.0)

        na = jnp.sqrt(jnp.sum(ha * ha, axis=1, keepdims=True))
        ne = jnp.sqrt(jnp.sum(he * he, axis=1, keepdims=True))
        cdims = (((1,), (1,)), ((), ()))
        gpp = lax.dot_general(ha, ha, cdims,
                              preferred_element_type=jnp.float32)
        gpe = lax.dot_general(ha, he, cdims,
                              preferred_element_type=jnp.float32)
        nna = lax.dot_general(na, na, cdims,
                              preferred_element_type=jnp.float32)
        nne = lax.dot_general(na, ne, cdims,
                              preferred_element_type=jnp.float32)
        sim_p = 1.0 - gpp / jnp.maximum(nna, 1e-8)
        dist_n = 1.0 - gpe / jnp.maximum(nne, 1e-8)

        posm = jnp.any(ha != 0.0, axis=1, keepdims=True).astype(jnp.float32)
        negm = jnp.any(he != 0.0, axis=1, keepdims=True).astype(jnp.float32)
        pos_num = jnp.maximum(jnp.sum(posm) - 1.0, 1.0)
        neg_cnt = jnp.sum(negm)
        neg_sample = jnp.dot(dist_n, negm,
                             preferred_element_type=jnp.float32)
        neg_sample = neg_sample / jnp.maximum(neg_cnt, 1.0)
        pos_sample = jnp.sum(sim_p, axis=1, keepdims=True) / pos_num
        li = jnp.maximum(pos_sample - neg_sample + margin, 0.0)
        active = posm * (neg_cnt > 0.0).astype(jnp.float32)
        closs = jnp.sum(li * active) / b
        closs_ref[...] = jnp.broadcast_to(closs, (1, 1))

        cw_v = cw_ref[...]
        logits = (jnp.dot(hg_ref[...], cw_v[:D, :],
                          preferred_element_type=jnp.float32)
                  + jnp.dot(ha, cw_v[D:, :],
                            preferred_element_type=jnp.float32)
                  + cb_ref[...])
        logits_ref[...] = logits

    return pl.pallas_call(
        body,
        out_shape=[
            jax.ShapeDtypeStruct((b, 1), jnp.float32),
            jax.ShapeDtypeStruct((1, 1), jnp.float32),
            jax.ShapeDtypeStruct((s, 1), jnp.float32),
        ],
    )(hg, hs, smf, mw1, mb1, mw2, mb2, cw, cb)


def _gnn(h, edge_index, layers, n_pad, e_per_worker, full_range, block_m):
    e = edge_index.shape[1]
    total = e_per_worker * (2 * NTILES if full_range else NTILES)
    ei = edge_index.astype(jnp.int32)
    src = jnp.concatenate([ei[0], jnp.zeros((total - e,), jnp.int32)])
    dst = jnp.concatenate([ei[1], jnp.full((total - e,), n_pad, jnp.int32)])

    segsum = _make_sc_segsum(n_pad, e_per_worker, full_range)
    nl = len(layers)
    for i, p in enumerate(layers):
        agg = segsum(h, src, dst)
        aggs = [agg[0], agg[1]] if full_range else [agg]
        h = _gin_mlp(h, aggs, p["W1"], p["b1"].reshape(1, -1), p["W2"],
                     p["b2"].reshape(1, -1), p["eps"].reshape(1, 1),
                     relu_out=(i < nl - 1), block_m=block_m)
    return h


def kernel(x, edge_index, batch, sub_x, sub_edge_index, sub_batch, sub_mask,
           params):
    n, d = x.shape
    ns = sub_x.shape[0]
    e = edge_index.shape[1]
    es = sub_edge_index.shape[1]
    b, s = sub_mask.shape

    n_pad = _round_up(n, 2048)
    ns_pad = _round_up(ns, 2048)
    e_per_worker = _round_up(-(-e // (2 * NTILES)), 16 * CHUNK)
    es_per_worker = _round_up(-(-es // NTILES), 8 * CHUNK)

    h0 = jnp.concatenate([x, jnp.zeros((n_pad - n, d), jnp.float32)])
    hs0 = jnp.concatenate([sub_x, jnp.zeros((ns_pad - ns, d), jnp.float32)])
    batch2d = jnp.concatenate(
        [batch, jnp.full((n_pad - n,), b, batch.dtype)]).reshape(n_pad, 1)
    sub_batch2d = jnp.concatenate(
        [sub_batch, jnp.full((ns_pad - ns,), s,
                             sub_batch.dtype)]).reshape(ns_pad, 1)
    smf = sub_mask.astype(jnp.float32)

    h = _gnn(h0, edge_index, params["gnn"], n_pad, e_per_worker,
             full_range=True, block_m=2048)
    hsub = _gnn(hs0, sub_edge_index, params["sub_gnn"], ns_pad,
                es_per_worker, full_range=False, block_m=2048)

    hg = _mean_pool(h, batch2d, b, block_m=512)
    hs_pool = _mean_pool(hsub, sub_batch2d, s, block_m=512)

    logits, closs, sg = _tail(
        hg, hs_pool, smf, params["mW1"], params["mb1"].reshape(1, -1),
        params["mW2"], params["mb2"].reshape(1, 1), params["cW"],
        params["cb"].reshape(1, 1), threshold=0.4, margin=1.0)
    return logits, closs.reshape(()), sg.reshape(s)


# speedup vs baseline: 1.1207x; 1.1207x over previous
import functools

import jax
import jax.numpy as jnp
from jax import lax
from jax.experimental import pallas as pl
from jax.experimental.pallas import tpu as pltpu
from jax.experimental.pallas import tpu_sc as plsc

D = 128
LANES = 16
NTILES = 16
CHUNK = 128
GCHUNKS = 8
GEDGES = GCHUNKS * CHUNK


def _round_up(v, m):
    return (v + m - 1) // m * m


@functools.lru_cache(maxsize=None)
def _make_sc_segsum(n_pad, e_per_worker, full_range):
    nh = n_pad if full_range else n_pad // 2
    acc_rows = nh + 8
    nzc = nh // CHUNK
    rpt = nh // NTILES
    nchunks = e_per_worker // CHUNK
    out_shape = ((2, n_pad, D) if full_range else (n_pad, D))
    mesh = plsc.VectorSubcoreMesh(core_axis_name="c", subcore_axis_name="s")

    @functools.partial(
        pl.kernel,
        mesh=mesh,
        out_type=jax.ShapeDtypeStruct(out_shape, jnp.float32),
        scratch_types=[
            pltpu.VMEM((e_per_worker,), jnp.int32),
            pltpu.VMEM((e_per_worker,), jnp.int32),
            pltpu.VMEM((CHUNK,), jnp.int32),
            pltpu.VMEM((CHUNK, D), jnp.float32),
            pltpu.VMEM_SHARED((acc_rows, D), jnp.float32),
            pltpu.SemaphoreType.DMA,
        ],
    )
    def segsum(h_hbm, src_hbm, dst_hbm, out_hbm, src_v, dst_v, didx, rows_v,
               acc, sem):
        c = lax.axis_index("c")
        s = lax.axis_index("s")
        base = c * nh

        def zero_row(r, carry):
            for i in range(D // LANES):
                rows_v[r, pl.ds(i * LANES, LANES)] = jnp.zeros(
                    (LANES,), jnp.float32)
            return carry

        lax.fori_loop(0, CHUNK, zero_row, 0)

        def zero_acc(jj, carry):
            j = s + jj * NTILES

            @pl.when(j < nzc)
            def _():
                pltpu.sync_copy(rows_v, acc.at[pl.ds(j * CHUNK, CHUNK)])

            return carry

        lax.fori_loop(0, (nzc + NTILES - 1) // NTILES, zero_acc, 0)

        w = (c * NTILES + s) if full_range else s
        ebase = w * e_per_worker
        pltpu.sync_copy(src_hbm.at[pl.ds(ebase, e_per_worker)], src_v)
        pltpu.sync_copy(dst_hbm.at[pl.ds(ebase, e_per_worker)], dst_v)
        plsc.subcore_barrier()

        def chunk_body(ch, carry):
            eoff = ch * CHUNK
            pltpu.async_copy(
                h_hbm.at[src_v.at[pl.ds(eoff, CHUNK)]], rows_v, sem).wait()
            for i in range(CHUNK // LANES):
                d_vec = dst_v[pl.ds(eoff + i * LANES, LANES)]
                if full_range:
                    didx[pl.ds(i * LANES, LANES)] = d_vec
                else:
                    dl = d_vec - base
                    ok = (dl >= 0) & (dl < nh)
                    didx[pl.ds(i * LANES, LANES)] = jnp.where(ok, dl, nh)
            pltpu.sync_copy(rows_v, acc.at[didx], add=True)
            return carry

        lax.fori_loop(0, nchunks, chunk_body, 0)
        plsc.subcore_barrier()

        if full_range:
            pltpu.sync_copy(acc.at[pl.ds(s * rpt, rpt)],
                            out_hbm.at[c, pl.ds(s * rpt, rpt)])
        else:
            pltpu.sync_copy(acc.at[pl.ds(s * rpt, rpt)],
                            out_hbm.at[pl.ds(c * nh + s * rpt, rpt)])

    return segsum


def _gin_mlp(h, aggs, w1, b1, w2, b2, eps, relu_out, block_m):
    n_pad = h.shape[0]
    d_h = w1.shape[1]
    naggs = len(aggs)

    def body(h_ref, *refs):
        a_refs = refs[:naggs]
        w1_ref, b1_ref, w2_ref, b2_ref, e_ref, o_ref = refs[naggs:]
        z = (1.0 + e_ref[0, 0]) * h_ref[...]
        for a_ref in a_refs:
            z = z + a_ref[...]
        z1 = jnp.dot(z, w1_ref[...], preferred_element_type=jnp.float32)
        z1 = jnp.maximum(z1 + b1_ref[...], 0.0)
        z2 = jnp.dot(z1, w2_ref[...], preferred_element_type=jnp.float32)
        z2 = z2 + b2_ref[...]
        if relu_out:
            z2 = jnp.maximum(z2, 0.0)
        o_ref[...] = z2

    return pl.pallas_call(
        body,
        grid=(n_pad // block_m,),
        in_specs=[pl.BlockSpec((block_m, D), lambda i: (i, 0))] * (1 + naggs)
        + [
            pl.BlockSpec((D, d_h), lambda i: (0, 0)),
            pl.BlockSpec((1, d_h), lambda i: (0, 0)),
            pl.BlockSpec((d_h, D), lambda i: (0, 0)),
            pl.BlockSpec((1, D), lambda i: (0, 0)),
            pl.BlockSpec((1, 1), lambda i: (0, 0)),
        ],
        out_specs=pl.BlockSpec((block_m, D), lambda i: (i, 0)),
        out_shape=jax.ShapeDtypeStruct((n_pad, D), jnp.float32),
    )(h, *aggs, w1, b1, w2, b2, eps)


def _mean_pool(h, seg2d, nseg, block_m):
    n_pad = h.shape[0]
    nsteps = n_pad // block_m

    def body(h_ref, s_ref, sum_ref, cnt_ref):
        i = pl.program_id(0)

        @pl.when(i == 0)
        def _():
            sum_ref[...] = jnp.zeros_like(sum_ref)
            cnt_ref[...] = jnp.zeros_like(cnt_ref)

        onehot = (s_ref[...] == lax.broadcasted_iota(
            jnp.int32, (block_m, nseg), 1)).astype(jnp.float32)
        sum_ref[...] += lax.dot_general(
            onehot, h_ref[...], (((0,), (0,)), ((), ())),
            preferred_element_type=jnp.float32)
        cnt_ref[...] += lax.dot_general(
            onehot, jnp.ones((block_m, 1), jnp.float32),
            (((0,), (0,)), ((), ())), preferred_element_type=jnp.float32)

        @pl.when(i == nsteps - 1)
        def _():
            sum_ref[...] = sum_ref[...] / jnp.maximum(cnt_ref[...], 1.0)

    mean, _ = pl.pallas_call(
        body,
        grid=(nsteps,),
        in_specs=[
            pl.BlockSpec((block_m, D), lambda i: (i, 0)),
            pl.BlockSpec((block_m, 1), lambda i: (i, 0)),
        ],
        out_specs=[
            pl.BlockSpec((nseg, D), lambda i: (0, 0)),
            pl.BlockSpec((nseg, 1), lambda i: (0, 0)),
        ],
        out_shape=[
            jax.ShapeDtypeStruct((nseg, D), jnp.float32),
            jax.ShapeDtypeStruct((nseg, 1), jnp.float32),
        ],
    )(h, seg2d)
    return mean


def _tail(hg, hs, smf, mw1, mb1, mw2, mb2, cw, cb, threshold, margin):
    b, s = smf.shape

    def body(hg_ref, hs_ref, smf_ref, mw1_ref, mb1_ref, mw2_ref, mb2_ref,
             cw_ref, cb_ref, logits_ref, closs_ref, sg_ref):
        hs_v = hs_ref[...]
        m1 = jnp.dot(hs_v, mw1_ref[...], preferred_element_type=jnp.float32)
        m1 = jnp.maximum(m1 + mb1_ref[...], 0.0)
        m = jnp.dot(m1, mw2_ref[...], preferred_element_type=jnp.float32)
        m = m + mb2_ref[...]
        sg = 1.0 / (1.0 + jnp.exp(-m))
        sg_ref[...] = sg

        smf_v = smf_ref[...]
        vmask = (sg > threshold).astype(jnp.float32)
        emask = (sg <= threshold - 0.1).astype(jnp.float32)
        ha_num = jnp.dot(smf_v, vmask * hs_v,
                         preferred_element_type=jnp.float32)
        da = jnp.dot(smf_v, vmask, preferred_element_type=jnp.float32)
        ha = ha_num / jnp.maximum(da, 1.0)
        he_num = jnp.dot(smf_v, emask * hs_v,
                         preferred_element_type=jnp.float32)
        de = jnp.dot(smf_v, emask, preferred_element_type=jnp.float32)
        he = he_num / jnp.maximum(de, 1.0)

        na = jnp.sqrt(jnp.sum(ha * ha, axis=1, keepdims=True))
        ne = jnp.sqrt(jnp.sum(he * he, axis=1, keepdims=True))
        cdims = (((1,), (1,)), ((), ()))
        gpp = lax.dot_general(ha, ha, cdims,
                              preferred_element_type=jnp.float32)
        gpe = lax.dot_general(ha, he, cdims,
                              preferred_element_type=jnp.float32)
        nna = lax.dot_general(na, na, cdims,
                              preferred_element_type=jnp.float32)
        nne = lax.dot_general(na, ne, cdims,
                              preferred_element_type=jnp.float32)
        sim_p = 1.0 - gpp / jnp.maximum(nna, 1e-8)
        dist_n = 1.0 - gpe / jnp.maximum(nne, 1e-8)

        posm = jnp.any(ha != 0.0, axis=1, keepdims=True).astype(jnp.float32)
        negm = jnp.any(he != 0.0, axis=1, keepdims=True).astype(jnp.float32)
        pos_num = jnp.maximum(jnp.sum(posm) - 1.0, 1.0)
        neg_cnt = jnp.sum(negm)
        neg_sample = jnp.dot(dist_n, negm,
                             preferred_element_type=jnp.float32)
        neg_sample = neg_sample / jnp.maximum(neg_cnt, 1.0)
        pos_sample = jnp.sum(sim_p, axis=1, keepdims=True) / pos_num
        li = jnp.maximum(pos_sample - neg_sample + margin, 0.0)
        active = posm * (neg_cnt > 0.0).astype(jnp.float32)
        closs = jnp.sum(li * active) / b
        closs_ref[...] = jnp.broadcast_to(closs, (1, 1))

        cw_v = cw_ref[...]
        logits = (jnp.dot(hg_ref[...], cw_v[:D, :],
                          preferred_element_type=jnp.float32)
                  + jnp.dot(ha, cw_v[D:, :],
                            preferred_element_type=jnp.float32)
                  + cb_ref[...])
        logits_ref[...] = logits

    return pl.pallas_call(
        body,
        out_shape=[
            jax.ShapeDtypeStruct((b, 1), jnp.float32),
            jax.ShapeDtypeStruct((1, 1), jnp.float32),
            jax.ShapeDtypeStruct((s, 1), jnp.float32),
        ],
    )(hg, hs, smf, mw1, mb1, mw2, mb2, cw, cb)


def _gnn(h, edge_index, layers, n_pad, e_per_worker, full_range, block_m):
    e = edge_index.shape[1]
    total = e_per_worker * (2 * NTILES if full_range else NTILES)
    ei = edge_index.astype(jnp.int32)
    src = jnp.concatenate([ei[0], jnp.zeros((total - e,), jnp.int32)])
    dst = jnp.concatenate([ei[1], jnp.full((total - e,), n_pad, jnp.int32)])

    segsum = _make_sc_segsum(n_pad, e_per_worker, full_range)
    nl = len(layers)
    for i, p in enumerate(layers):
        agg = segsum(h, src, dst)
        aggs = [agg[0], agg[1]] if full_range else [agg]
        h = _gin_mlp(h, aggs, p["W1"], p["b1"].reshape(1, -1), p["W2"],
                     p["b2"].reshape(1, -1), p["eps"].reshape(1, 1),
                     relu_out=(i < nl - 1), block_m=block_m)
    return h


def kernel(x, edge_index, batch, sub_x, sub_edge_index, sub_batch, sub_mask,
           params):
    n, d = x.shape
    ns = sub_x.shape[0]
    e = edge_index.shape[1]
    es = sub_edge_index.shape[1]
    b, s = sub_mask.shape

    n_pad = _round_up(n, 2048)
    ns_pad = _round_up(ns, 2048)
    e_per_worker = _round_up(-(-e // NTILES), CHUNK)
    es_per_worker = _round_up(-(-es // NTILES), CHUNK)

    h0 = jnp.concatenate([x, jnp.zeros((n_pad - n, d), jnp.float32)])
    hs0 = jnp.concatenate([sub_x, jnp.zeros((ns_pad - ns, d), jnp.float32)])
    batch2d = jnp.concatenate(
        [batch, jnp.full((n_pad - n,), b, batch.dtype)]).reshape(n_pad, 1)
    sub_batch2d = jnp.concatenate(
        [sub_batch, jnp.full((ns_pad - ns,), s,
                             sub_batch.dtype)]).reshape(ns_pad, 1)
    smf = sub_mask.astype(jnp.float32)

    h = _gnn(h0, edge_index, params["gnn"], n_pad, e_per_worker,
             full_range=False, block_m=2048)
    hsub = _gnn(hs0, sub_edge_index, params["sub_gnn"], ns_pad,
                es_per_worker, full_range=False, block_m=2048)

    hg = _mean_pool(h, batch2d, b, block_m=512)
    hs_pool = _mean_pool(hsub, sub_batch2d, s, block_m=512)

    logits, closs, sg = _tail(
        hg, hs_pool, smf, params["mW1"], params["mb1"].reshape(1, -1),
        params["mW2"], params["mb2"].reshape(1, 1), params["cW"],
        params["cb"].reshape(1, 1), threshold=0.4, margin=1.0)
    return logits, closs.reshape(()), sg.reshape(s)
